# baseline (speedup 1.0000x reference)
import functools

import jax
import jax.numpy as jnp
from jax import lax
from jax.experimental import pallas as pl
from jax.experimental.pallas import tpu as pltpu
from jax.experimental.pallas import tpu_sc as plsc

NUM_EMB = 1024
EMB_DIM = 256
ROWS = 16 * 1024
TILE = 1024

NUM_WORKERS = 32
ROWS_PER_W = ROWS // NUM_WORKERS
CHUNK = 128
NCHUNK = ROWS_PER_W // CHUNK


def _tc_body(x_ref, e_ref, x2_ref, e2_ref, enc_ref, idx_ref):
    x = x_ref[:]
    emb = e_ref[:]
    x2 = x2_ref[:]
    e2 = e2_ref[0, 0, :]
    nsim2 = jnp.dot(-2.0 * x, emb, preferred_element_type=jnp.float32)
    dist = (x2 + e2[None, :]) + nsim2
    m = jnp.min(dist, axis=1, keepdims=True)
    iota_f = lax.broadcasted_iota(jnp.int32, dist.shape, 1).astype(jnp.float32)
    nmin = jnp.min(jnp.where(dist == m, iota_f, 1.0e9), axis=1, keepdims=True)
    enc_ref[:] = (iota_f == nmin).astype(jnp.float32)
    idx_ref[0, 0, :] = nmin[:, 0].astype(jnp.int32)


@functools.cache
def _sc_gather_kernel():
    mesh = plsc.VectorSubcoreMesh(core_axis_name="c", subcore_axis_name="s")

    @functools.partial(
        pl.kernel,
        mesh=mesh,
        out_type=jax.ShapeDtypeStruct((ROWS, EMB_DIM), jnp.float32),
        scratch_types=[
            pltpu.VMEM((CHUNK,), jnp.int32),
            pltpu.VMEM((CHUNK, EMB_DIM), jnp.float32),
            pltpu.SemaphoreType.DMA,
        ],
    )
    def _sc_gather(table_hbm, idx_hbm, out_hbm, idx_v, rows_v, sem):
        wid = lax.axis_index("s") * 2 + lax.axis_index("c")
        base = wid * ROWS_PER_W
        for c in range(NCHUNK):
            off = base + c * CHUNK
            pltpu.sync_copy(idx_hbm.at[pl.ds(off, CHUNK)], idx_v)
            pltpu.async_copy(table_hbm.at[idx_v], rows_v, sem).wait()
            pltpu.sync_copy(rows_v, out_hbm.at[pl.ds(off, CHUNK)])

    return _sc_gather


def kernel(batch, embeddings):
    input_shape = batch.shape
    flat = jnp.reshape(batch, (ROWS, EMB_DIM))
    x2 = jnp.sum(flat ** 2, axis=1, keepdims=True)
    e2 = jnp.reshape(jnp.sum(embeddings ** 2, axis=0), (1, 1, NUM_EMB))
    grid = ROWS // TILE
    enc, idx3 = pl.pallas_call(
        _tc_body,
        grid=(grid,),
        in_specs=[
            pl.BlockSpec((TILE, EMB_DIM), lambda i: (i, 0)),
            pl.BlockSpec((EMB_DIM, NUM_EMB), lambda i: (0, 0)),
            pl.BlockSpec((TILE, 1), lambda i: (i, 0)),
            pl.BlockSpec((1, 1, NUM_EMB), lambda i: (0, 0, 0)),
        ],
        out_specs=[
            pl.BlockSpec((TILE, NUM_EMB), lambda i: (i, 0)),
            pl.BlockSpec((1, 1, TILE), lambda i: (i, 0, 0)),
        ],
        out_shape=[
            jax.ShapeDtypeStruct((ROWS, NUM_EMB), jnp.float32),
            jax.ShapeDtypeStruct((grid, 1, TILE), jnp.int32),
        ],
    )(flat, embeddings, x2, e2)
    table = embeddings.T
    quant = _sc_gather_kernel()(table, jnp.reshape(idx3, (ROWS,)))
    return (enc, jnp.reshape(quant, input_shape))


# speedup vs baseline: 1.5754x; 1.5754x over previous
import jax
import jax.numpy as jnp
from jax import lax
from jax.experimental import pallas as pl

NUM_EMB = 1024
EMB_DIM = 256
ROWS = 16 * 1024
TILE = 1024


def _vq_body(x_ref, e_ref, x2_ref, e2_ref, enc_ref, out_ref):
    x = x_ref[:]
    emb = e_ref[:]
    x2 = x2_ref[:]
    e2 = e2_ref[0, 0, :]
    nsim2 = jnp.dot(-2.0 * x, emb, preferred_element_type=jnp.float32)
    dist = (x2 + e2[None, :]) + nsim2
    m = jnp.min(dist, axis=1, keepdims=True)
    iota_f = lax.broadcasted_iota(jnp.int32, dist.shape, 1).astype(jnp.float32)
    nmin = jnp.min(jnp.where(dist == m, iota_f, 1.0e9), axis=1, keepdims=True)
    onehot = (iota_f == nmin).astype(jnp.float32)
    enc_ref[:] = onehot
    out_ref[:] = jax.lax.dot_general(
        onehot, emb, (((1,), (1,)), ((), ())),
        preferred_element_type=jnp.float32)


def kernel(batch, embeddings):
    input_shape = batch.shape
    flat = jnp.reshape(batch, (ROWS, EMB_DIM))
    x2 = jnp.sum(flat ** 2, axis=1, keepdims=True)
    e2 = jnp.reshape(jnp.sum(embeddings ** 2, axis=0), (1, 1, NUM_EMB))
    grid = ROWS // TILE
    enc, quant = pl.pallas_call(
        _vq_body,
        grid=(grid,),
        in_specs=[
            pl.BlockSpec((TILE, EMB_DIM), lambda i: (i, 0)),
            pl.BlockSpec((EMB_DIM, NUM_EMB), lambda i: (0, 0)),
            pl.BlockSpec((TILE, 1), lambda i: (i, 0)),
            pl.BlockSpec((1, 1, NUM_EMB), lambda i: (0, 0, 0)),
        ],
        out_specs=[
            pl.BlockSpec((TILE, NUM_EMB), lambda i: (i, 0)),
            pl.BlockSpec((TILE, EMB_DIM), lambda i: (i, 0)),
        ],
        out_shape=[
            jax.ShapeDtypeStruct((ROWS, NUM_EMB), jnp.float32),
            jax.ShapeDtypeStruct((ROWS, EMB_DIM), jnp.float32),
        ],
    )(flat, embeddings, x2, e2)
    return (enc, jnp.reshape(quant, input_shape))


# speedup vs baseline: 1.6596x; 1.0535x over previous
import jax
import jax.numpy as jnp
from jax import lax
from jax.experimental import pallas as pl

NUM_EMB = 1024
EMB_DIM = 256
ROWS = 16 * 1024
TILE = 2048


def _vq_body(x_ref, e_ref, x2_ref, e2_ref, enc_ref, out_ref):
    x = x_ref[:]
    emb = e_ref[:]
    x2 = x2_ref[:]
    e2 = e2_ref[0, 0, :]
    nsim2 = jnp.dot(-2.0 * x, emb, preferred_element_type=jnp.float32)
    dist = (x2 + e2[None, :]) + nsim2
    m = jnp.min(dist, axis=1, keepdims=True)
    iota_f = lax.broadcasted_iota(jnp.int32, dist.shape, 1).astype(jnp.float32)
    nmin = jnp.min(jnp.where(dist == m, iota_f, 1.0e9), axis=1, keepdims=True)
    onehot = (iota_f == nmin).astype(jnp.float32)
    enc_ref[:] = onehot
    out_ref[:] = jax.lax.dot_general(
        onehot, emb, (((1,), (1,)), ((), ())),
        preferred_element_type=jnp.float32)


def kernel(batch, embeddings):
    input_shape = batch.shape
    flat = jnp.reshape(batch, (ROWS, EMB_DIM))
    x2 = jnp.sum(flat ** 2, axis=1, keepdims=True)
    e2 = jnp.reshape(jnp.sum(embeddings ** 2, axis=0), (1, 1, NUM_EMB))
    grid = ROWS // TILE
    enc, quant = pl.pallas_call(
        _vq_body,
        grid=(grid,),
        in_specs=[
            pl.BlockSpec((TILE, EMB_DIM), lambda i: (i, 0)),
            pl.BlockSpec((EMB_DIM, NUM_EMB), lambda i: (0, 0)),
            pl.BlockSpec((TILE, 1), lambda i: (i, 0)),
            pl.BlockSpec((1, 1, NUM_EMB), lambda i: (0, 0, 0)),
        ],
        out_specs=[
            pl.BlockSpec((TILE, NUM_EMB), lambda i: (i, 0)),
            pl.BlockSpec((TILE, EMB_DIM), lambda i: (i, 0)),
        ],
        out_shape=[
            jax.ShapeDtypeStruct((ROWS, NUM_EMB), jnp.float32),
            jax.ShapeDtypeStruct((ROWS, EMB_DIM), jnp.float32),
        ],
    )(flat, embeddings, x2, e2)
    return (enc, jnp.reshape(quant, input_shape))
